# baseline (speedup 1.0000x reference)
import functools

import jax
import jax.numpy as jnp
from jax import lax
from jax.experimental import pallas as pl
from jax.experimental.pallas import tpu as pltpu
from jax.experimental.pallas import tpu_sc as plsc

_SCORE_THRESH = 0.05
_NMS_THRESH = 0.5
_MAX_DET = 100
_L = 16
_NS = 16

_GDN = lax.GatherDimensionNumbers(
    offset_dims=(), collapsed_slice_dims=(0,), start_index_map=(0,))


def _permute(v, idx):
    return lax.gather(v, idx[:, None], _GDN, (1,),
                      mode=lax.GatherScatterMode.PROMISE_IN_BOUNDS)


def _bcast_max(v, lanei):
    for k in (8, 4, 2, 1):
        v = jnp.maximum(v, _permute(v, lanei ^ k))
    return v


def _nms_sc_body(x1h, y1h, x2h, y2h, sh, outh,
                 x1v, y1v, x2v, y2v, av, sv,
                 stage, loc_pack, out_stage, sh_pack, sem):
    npad = x1h.shape[0]
    per = npad // _NS
    cid = lax.axis_index("c")
    tid = lax.axis_index("s")
    base = tid * per

    cps = [pltpu.async_copy(h.at[pl.ds(base, per)], v, sem)
           for h, v in ((x1h, x1v), (y1h, y1v), (x2h, x2v), (y2h, y2v),
                        (sh, sv))]
    for cp in cps:
        cp.wait()

    lanei = lax.iota(jnp.int32, _L)
    zi = jnp.zeros((_L,), jnp.int32)
    neg = jnp.float32(-3.4e38)

    @plsc.parallel_loop(0, per, _L, unroll=4)
    def _(i):
        sl = pl.ds(i, _L)
        s = sv[sl]
        sv[sl] = jnp.where(s > _SCORE_THRESH, s, -1.0)
        av[sl] = (x2v[sl] - x1v[sl]) * (y2v[sl] - y1v[sl])

    def round_body(r, carry):
        bx1, by1, bx2, by2, ba = carry

        acc0 = (jnp.full((_L,), neg, jnp.float32),
                jnp.full((_L,), 2 ** 30, jnp.int32))

        @plsc.parallel_loop(0, per, 4 * _L, unroll=1,
                            carry=(acc0, acc0, acc0, acc0))
        def pass_out(i, accs):
            out = []
            for k in range(4):
                accv, acci = accs[k]
                sl = pl.ds(i + k * _L, _L)
                xx1 = x1v[sl]
                yy1 = y1v[sl]
                xx2 = x2v[sl]
                yy2 = y2v[sl]
                ar = av[sl]
                s = sv[sl]
                w = jnp.maximum(
                    jnp.minimum(bx2, xx2) - jnp.maximum(bx1, xx1), 0.0)
                h = jnp.maximum(
                    jnp.minimum(by2, yy2) - jnp.maximum(by1, yy1), 0.0)
                inter = w * h
                iou = inter / (ba + ar - inter + 1e-9)
                s = jnp.where(iou > _NMS_THRESH, -1.0, s)
                sv[sl] = s
                idx = i + (k * _L + lanei)
                upd = (s > accv) | ((s == accv) & (idx < acci))
                accv = jnp.where(upd, s, accv)
                acci = jnp.where(upd, idx, acci)
                out.append((accv, acci))
            return tuple(out)

        (accv, acci) = pass_out[0]
        for k in range(1, 4):
            v2, i2 = pass_out[k]
            better = (v2 > accv) | ((v2 == accv) & (i2 < acci))
            accv = jnp.where(better, v2, accv)
            acci = jnp.where(better, i2, acci)
        mvec = _bcast_max(accv, lanei)
        limin = jnp.where(accv >= mvec, acci, jnp.int32(2 ** 30))
        for k in (8, 4, 2, 1):
            limin = jnp.minimum(limin, _permute(limin, lanei ^ k))
        cx1 = plsc.load_gather(x1v, [limin])
        cy1 = plsc.load_gather(y1v, [limin])
        cx2 = plsc.load_gather(x2v, [limin])
        cy2 = plsc.load_gather(y2v, [limin])
        car = (cx2 - cx1) * (cy2 - cy1)
        pack = jnp.where(lanei == 0, cx1,
               jnp.where(lanei == 1, cy1,
               jnp.where(lanei == 2, cx2,
               jnp.where(lanei == 3, cy2,
               jnp.where(lanei == 4, mvec,
               jnp.where(lanei == 5, car, 0.0))))))
        stage[...] = pack
        off = (r & 1) * (_NS * _L)
        pltpu.sync_copy(stage, sh_pack.at[pl.ds(off + tid * _L, _L)])
        plsc.subcore_barrier()
        pltpu.sync_copy(sh_pack.at[pl.ds(off, _NS * _L)], loc_pack)

        g = loc_pack[pl.ds(0, _L)]
        gv = _permute(g, zi + 4)
        for t in range(1, _NS):
            pt = loc_pack[pl.ds(t * _L, _L)]
            vt = _permute(pt, zi + 4)
            upd = vt > gv
            gv = jnp.where(upd, vt, gv)
            g = jnp.where(upd, pt, g)

        nbx1 = _permute(g, zi)
        nby1 = _permute(g, zi + 1)
        nbx2 = _permute(g, zi + 2)
        nby2 = _permute(g, zi + 3)
        nba = (nbx2 - nbx1) * (nby2 - nby1)

        rowv = jnp.where((lanei <= 4) & (gv > 0.0), g, 0.0)
        out_stage[pl.ds(r * _L, _L)] = rowv

        return nbx1, nby1, nbx2, nby2, nba

    z = jnp.zeros((_L,), jnp.float32)
    lax.fori_loop(0, _MAX_DET, round_body, (z, z, z, z, z))

    @pl.when(jnp.logical_and(cid == 0, tid == 0))
    def _():
        pltpu.sync_copy(out_stage, outh)


def kernel(boxes, scores):
    n = boxes.shape[0]
    npad = ((n + _NS * _L - 1) // (_NS * _L)) * (_NS * _L)
    pad = npad - n
    b = jnp.pad(boxes, ((0, pad), (0, 0)))
    s = jnp.pad(scores, (0, pad))
    per = npad // _NS
    mesh = plsc.VectorSubcoreMesh(core_axis_name="c", subcore_axis_name="s")
    f = functools.partial(
        pl.kernel,
        out_type=jax.ShapeDtypeStruct((_MAX_DET * _L,), jnp.float32),
        mesh=mesh,
        compiler_params=pltpu.CompilerParams(needs_layout_passes=False),
        scratch_types=[
            pltpu.VMEM((per,), jnp.float32),
            pltpu.VMEM((per,), jnp.float32),
            pltpu.VMEM((per,), jnp.float32),
            pltpu.VMEM((per,), jnp.float32),
            pltpu.VMEM((per,), jnp.float32),
            pltpu.VMEM((per,), jnp.float32),
            pltpu.VMEM((_L,), jnp.float32),
            pltpu.VMEM((_NS * _L,), jnp.float32),
            pltpu.VMEM((_MAX_DET * _L,), jnp.float32),
            pltpu.VMEM_SHARED((2 * _NS * _L,), jnp.float32),
            pltpu.SemaphoreType.DMA,
        ],
    )(_nms_sc_body)
    out = f(b[:, 0], b[:, 1], b[:, 2], b[:, 3], s)
    return out.reshape(_MAX_DET, _L)[:, :5]


# speedup vs baseline: 1.0034x; 1.0034x over previous
import functools

import jax
import jax.numpy as jnp
from jax import lax
from jax.experimental import pallas as pl
from jax.experimental.pallas import tpu as pltpu
from jax.experimental.pallas import tpu_sc as plsc

_SCORE_THRESH = 0.05
_NMS_THRESH = 0.5
_MAX_DET = 100
_L = 16
_NS = 16

_GDN = lax.GatherDimensionNumbers(
    offset_dims=(), collapsed_slice_dims=(0,), start_index_map=(0,))


def _permute(v, idx):
    return lax.gather(v, idx[:, None], _GDN, (1,),
                      mode=lax.GatherScatterMode.PROMISE_IN_BOUNDS)


def _bcast_max(v, lanei):
    for k in (8, 4, 2, 1):
        v = jnp.maximum(v, _permute(v, lanei ^ k))
    return v


def _nms_sc_body(x1h, y1h, x2h, y2h, sh, outh,
                 x1v, y1v, x2v, y2v, av, sv,
                 stage, loc_pack, out_stage, sh_pack, sem):
    npad = x1h.shape[0]
    per = npad // _NS
    cid = lax.axis_index("c")
    tid = lax.axis_index("s")
    base = tid * per

    cps = [pltpu.async_copy(h.at[pl.ds(base, per)], v, sem)
           for h, v in ((x1h, x1v), (y1h, y1v), (x2h, x2v), (y2h, y2v),
                        (sh, sv))]
    for cp in cps:
        cp.wait()

    lanei = lax.iota(jnp.int32, _L)
    zi = jnp.zeros((_L,), jnp.int32)
    neg = jnp.float32(-3.4e38)

    @plsc.parallel_loop(0, per, _L, unroll=4)
    def _(i):
        sl = pl.ds(i, _L)
        s = sv[sl]
        sv[sl] = jnp.where(s > _SCORE_THRESH, s, -1.0)
        av[sl] = (x2v[sl] - x1v[sl]) * (y2v[sl] - y1v[sl])

    def round_body(r, carry):
        bx1, by1, bx2, by2, ba = carry

        acc0 = (jnp.full((_L,), neg, jnp.float32),
                jnp.full((_L,), 2 ** 30, jnp.int32))

        @plsc.parallel_loop(0, per, 4 * _L, unroll=1,
                            carry=(acc0, acc0, acc0, acc0))
        def pass_out(i, accs):
            out = []
            for k in range(4):
                accv, acci = accs[k]
                sl = pl.ds(i + k * _L, _L)
                xx1 = x1v[sl]
                yy1 = y1v[sl]
                xx2 = x2v[sl]
                yy2 = y2v[sl]
                ar = av[sl]
                s = sv[sl]
                w = jnp.maximum(
                    jnp.minimum(bx2, xx2) - jnp.maximum(bx1, xx1), 0.0)
                h = jnp.maximum(
                    jnp.minimum(by2, yy2) - jnp.maximum(by1, yy1), 0.0)
                inter = w * h
                iou = inter / (ba + ar - inter + 1e-9)
                s = jnp.where(iou > _NMS_THRESH, -1.0, s)
                sv[sl] = s
                idx = i + (k * _L + lanei)
                upd = (s > accv) | ((s == accv) & (idx < acci))
                accv = jnp.where(upd, s, accv)
                acci = jnp.where(upd, idx, acci)
                out.append((accv, acci))
            return tuple(out)

        (accv, acci) = pass_out[0]
        for k in range(1, 4):
            v2, i2 = pass_out[k]
            better = (v2 > accv) | ((v2 == accv) & (i2 < acci))
            accv = jnp.where(better, v2, accv)
            acci = jnp.where(better, i2, acci)
        mvec = _bcast_max(accv, lanei)
        limin = jnp.where(accv >= mvec, acci, jnp.int32(2 ** 30))
        for k in (8, 4, 2, 1):
            limin = jnp.minimum(limin, _permute(limin, lanei ^ k))
        cx1 = plsc.load_gather(x1v, [limin])
        cy1 = plsc.load_gather(y1v, [limin])
        cx2 = plsc.load_gather(x2v, [limin])
        cy2 = plsc.load_gather(y2v, [limin])
        car = (cx2 - cx1) * (cy2 - cy1)
        pack = jnp.where(lanei == 0, cx1,
               jnp.where(lanei == 1, cy1,
               jnp.where(lanei == 2, cx2,
               jnp.where(lanei == 3, cy2,
               jnp.where(lanei == 4, mvec,
               jnp.where(lanei == 5, car, 0.0))))))
        stage[...] = pack
        off = (r & 1) * (_NS * _L)
        pltpu.sync_copy(stage, sh_pack.at[pl.ds(off + tid * _L, _L)])
        plsc.subcore_barrier()
        pltpu.sync_copy(sh_pack.at[pl.ds(off, _NS * _L)], loc_pack)

        rows = [loc_pack[pl.ds(t * _L, _L)] for t in range(_NS)]
        pairs = [(_permute(p, zi + 4), p) for p in rows]
        while len(pairs) > 1:
            nxt = []
            for a in range(0, len(pairs), 2):
                (vl, pl_), (vr, pr) = pairs[a], pairs[a + 1]
                upd = vr > vl
                nxt.append((jnp.where(upd, vr, vl),
                            jnp.where(upd, pr, pl_)))
            pairs = nxt
        gv, g = pairs[0]

        nbx1 = _permute(g, zi)
        nby1 = _permute(g, zi + 1)
        nbx2 = _permute(g, zi + 2)
        nby2 = _permute(g, zi + 3)
        nba = (nbx2 - nbx1) * (nby2 - nby1)

        rowv = jnp.where((lanei <= 4) & (gv > 0.0), g, 0.0)
        out_stage[pl.ds(r * _L, _L)] = rowv

        return nbx1, nby1, nbx2, nby2, nba

    z = jnp.zeros((_L,), jnp.float32)
    lax.fori_loop(0, _MAX_DET, round_body, (z, z, z, z, z))

    @pl.when(jnp.logical_and(cid == 0, tid == 0))
    def _():
        pltpu.sync_copy(out_stage, outh)


def kernel(boxes, scores):
    n = boxes.shape[0]
    npad = ((n + _NS * _L - 1) // (_NS * _L)) * (_NS * _L)
    pad = npad - n
    b = jnp.pad(boxes, ((0, pad), (0, 0)))
    s = jnp.pad(scores, (0, pad))
    per = npad // _NS
    mesh = plsc.VectorSubcoreMesh(core_axis_name="c", subcore_axis_name="s")
    f = functools.partial(
        pl.kernel,
        out_type=jax.ShapeDtypeStruct((_MAX_DET * _L,), jnp.float32),
        mesh=mesh,
        compiler_params=pltpu.CompilerParams(needs_layout_passes=False),
        scratch_types=[
            pltpu.VMEM((per,), jnp.float32),
            pltpu.VMEM((per,), jnp.float32),
            pltpu.VMEM((per,), jnp.float32),
            pltpu.VMEM((per,), jnp.float32),
            pltpu.VMEM((per,), jnp.float32),
            pltpu.VMEM((per,), jnp.float32),
            pltpu.VMEM((_L,), jnp.float32),
            pltpu.VMEM((_NS * _L,), jnp.float32),
            pltpu.VMEM((_MAX_DET * _L,), jnp.float32),
            pltpu.VMEM_SHARED((2 * _NS * _L,), jnp.float32),
            pltpu.SemaphoreType.DMA,
        ],
    )(_nms_sc_body)
    out = f(b[:, 0], b[:, 1], b[:, 2], b[:, 3], s)
    return out.reshape(_MAX_DET, _L)[:, :5]


# speedup vs baseline: 1.0694x; 1.0657x over previous
import functools

import jax
import jax.numpy as jnp
from jax import lax
from jax.experimental import pallas as pl
from jax.experimental.pallas import tpu as pltpu
from jax.experimental.pallas import tpu_sc as plsc

_SCORE_THRESH = 0.05
_NMS_THRESH = 0.5
_MAX_DET = 100
_L = 16
_NS = 16

_GDN = lax.GatherDimensionNumbers(
    offset_dims=(), collapsed_slice_dims=(0,), start_index_map=(0,))


def _permute(v, idx):
    return lax.gather(v, idx[:, None], _GDN, (1,),
                      mode=lax.GatherScatterMode.PROMISE_IN_BOUNDS)


def _bcast_max(v, lanei):
    for k in (8, 4, 2, 1):
        v = jnp.maximum(v, _permute(v, lanei ^ k))
    return v


def _nms_sc_body(x1h, y1h, x2h, y2h, sh, outh,
                 x1v, y1v, x2v, y2v, av, sv,
                 stage, loc_pack, out_stage, sh_pack, sem):
    npad = x1h.shape[0]
    per = npad // _NS
    cid = lax.axis_index("c")
    tid = lax.axis_index("s")
    base = tid * per

    cps = [pltpu.async_copy(h.at[pl.ds(base, per)], v.at[pl.ds(0, per)], sem)
           for h, v in ((x1h, x1v), (y1h, y1v), (x2h, x2v), (y2h, y2v),
                        (sh, sv))]
    for cp in cps:
        cp.wait()

    lanei = lax.iota(jnp.int32, _L)
    zi = jnp.zeros((_L,), jnp.int32)
    neg = jnp.float32(-3.4e38)

    @plsc.parallel_loop(0, per, _L, unroll=4)
    def _(i):
        sl = pl.ds(i, _L)
        s = sv[sl]
        sv[sl] = jnp.where(s > _SCORE_THRESH, s, -1.0)
        av[sl] = (x2v[sl] - x1v[sl]) * (y2v[sl] - y1v[sl])

    def round_body(r, carry):
        bx1, by1, bx2, by2, ba = carry

        acc0 = (jnp.full((_L,), neg, jnp.float32),
                jnp.full((_L,), 2 ** 30, jnp.int32))

        iz = r * 0
        accs = (acc0, acc0, acc0, acc0)
        for i in range(0, per, 4 * _L):
            out = []
            for k in range(4):
                accv, acci = accs[k]
                sl = pl.ds(iz + (i + k * _L), _L)
                xx1 = x1v[sl]
                yy1 = y1v[sl]
                xx2 = x2v[sl]
                yy2 = y2v[sl]
                ar = av[sl]
                s = sv[sl]
                w = jnp.maximum(
                    jnp.minimum(bx2, xx2) - jnp.maximum(bx1, xx1), 0.0)
                h = jnp.maximum(
                    jnp.minimum(by2, yy2) - jnp.maximum(by1, yy1), 0.0)
                inter = w * h
                iou = inter / (ba + ar - inter + 1e-9)
                s = jnp.where(iou > _NMS_THRESH, -1.0, s)
                sv[sl] = s
                idx = i + (k * _L + lanei)
                upd = (s > accv) | ((s == accv) & (idx < acci))
                accv = jnp.where(upd, s, accv)
                acci = jnp.where(upd, idx, acci)
                out.append((accv, acci))
            accs = tuple(out)

        (accv, acci) = accs[0]
        for k in range(1, 4):
            v2, i2 = accs[k]
            better = (v2 > accv) | ((v2 == accv) & (i2 < acci))
            accv = jnp.where(better, v2, accv)
            acci = jnp.where(better, i2, acci)
        mvec = _bcast_max(accv, lanei)
        limin = jnp.where(accv >= mvec, acci, jnp.int32(2 ** 30))
        for k in (8, 4, 2, 1):
            limin = jnp.minimum(limin, _permute(limin, lanei ^ k))
        cx1 = plsc.load_gather(x1v, [limin])
        cy1 = plsc.load_gather(y1v, [limin])
        cx2 = plsc.load_gather(x2v, [limin])
        cy2 = plsc.load_gather(y2v, [limin])
        car = (cx2 - cx1) * (cy2 - cy1)
        pack = jnp.where(lanei == 0, cx1,
               jnp.where(lanei == 1, cy1,
               jnp.where(lanei == 2, cx2,
               jnp.where(lanei == 3, cy2,
               jnp.where(lanei == 4, mvec,
               jnp.where(lanei == 5, car, 0.0))))))
        stage[...] = pack
        off = (r & 1) * (_NS * _L)
        pltpu.sync_copy(stage, sh_pack.at[pl.ds(off + tid * _L, _L)])
        plsc.subcore_barrier()
        pltpu.sync_copy(sh_pack.at[pl.ds(off, _NS * _L)], loc_pack)

        rows = [loc_pack[pl.ds(t * _L, _L)] for t in range(_NS)]
        pairs = [(_permute(p, zi + 4), p) for p in rows]
        while len(pairs) > 1:
            nxt = []
            for a in range(0, len(pairs), 2):
                (vl, pl_), (vr, pr) = pairs[a], pairs[a + 1]
                upd = vr > vl
                nxt.append((jnp.where(upd, vr, vl),
                            jnp.where(upd, pr, pl_)))
            pairs = nxt
        gv, g = pairs[0]

        nbx1 = _permute(g, zi)
        nby1 = _permute(g, zi + 1)
        nbx2 = _permute(g, zi + 2)
        nby2 = _permute(g, zi + 3)
        nba = (nbx2 - nbx1) * (nby2 - nby1)

        rowv = jnp.where((lanei <= 4) & (gv > 0.0), g, 0.0)
        out_stage[pl.ds(r * _L, _L)] = rowv

        return nbx1, nby1, nbx2, nby2, nba

    z = jnp.zeros((_L,), jnp.float32)
    lax.fori_loop(0, _MAX_DET, round_body, (z, z, z, z, z))

    @pl.when(jnp.logical_and(cid == 0, tid == 0))
    def _():
        pltpu.sync_copy(out_stage, outh)


def kernel(boxes, scores):
    n = boxes.shape[0]
    npad = ((n + _NS * _L - 1) // (_NS * _L)) * (_NS * _L)
    pad = npad - n
    b = jnp.pad(boxes, ((0, pad), (0, 0)))
    s = jnp.pad(scores, (0, pad))
    per = npad // _NS
    mesh = plsc.VectorSubcoreMesh(core_axis_name="c", subcore_axis_name="s")
    f = functools.partial(
        pl.kernel,
        out_type=jax.ShapeDtypeStruct((_MAX_DET * _L,), jnp.float32),
        mesh=mesh,
        compiler_params=pltpu.CompilerParams(needs_layout_passes=False),
        scratch_types=[
            pltpu.VMEM((per + 32,), jnp.float32),
            pltpu.VMEM((per + 32,), jnp.float32),
            pltpu.VMEM((per + 32,), jnp.float32),
            pltpu.VMEM((per + 32,), jnp.float32),
            pltpu.VMEM((per + 32,), jnp.float32),
            pltpu.VMEM((per + 32,), jnp.float32),
            pltpu.VMEM((_L,), jnp.float32),
            pltpu.VMEM((_NS * _L,), jnp.float32),
            pltpu.VMEM((_MAX_DET * _L,), jnp.float32),
            pltpu.VMEM_SHARED((2 * _NS * _L,), jnp.float32),
            pltpu.SemaphoreType.DMA,
        ],
    )(_nms_sc_body)
    out = f(b[:, 0], b[:, 1], b[:, 2], b[:, 3], s)
    return out.reshape(_MAX_DET, _L)[:, :5]


# speedup vs baseline: 1.0941x; 1.0232x over previous
import functools

import jax
import jax.numpy as jnp
from jax import lax
from jax.experimental import pallas as pl
from jax.experimental.pallas import tpu as pltpu
from jax.experimental.pallas import tpu_sc as plsc

_SCORE_THRESH = 0.05
_NMS_THRESH = 0.5
_MAX_DET = 100
_L = 16
_NS = 16

_GDN = lax.GatherDimensionNumbers(
    offset_dims=(), collapsed_slice_dims=(0,), start_index_map=(0,))


def _permute(v, idx):
    return lax.gather(v, idx[:, None], _GDN, (1,),
                      mode=lax.GatherScatterMode.PROMISE_IN_BOUNDS)


def _bcast_max(v, lanei):
    for k in (8, 4, 2, 1):
        v = jnp.maximum(v, _permute(v, lanei ^ k))
    return v


def _nms_sc_body(x1h, y1h, x2h, y2h, sh, outh,
                 x1v, y1v, x2v, y2v, av, sv,
                 stage, loc_pack, out_stage, sh_pack, sem):
    npad = x1h.shape[0]
    per = npad // _NS
    cid = lax.axis_index("c")
    tid = lax.axis_index("s")
    base = tid * per

    cps = [pltpu.async_copy(h.at[pl.ds(base, per)], v.at[pl.ds(0, per)], sem)
           for h, v in ((x1h, x1v), (y1h, y1v), (x2h, x2v), (y2h, y2v),
                        (sh, sv))]
    for cp in cps:
        cp.wait()

    lanei = lax.iota(jnp.int32, _L)
    zi = jnp.zeros((_L,), jnp.int32)
    neg = jnp.float32(-3.4e38)

    @plsc.parallel_loop(0, per, _L, unroll=4)
    def _(i):
        sl = pl.ds(i, _L)
        s = sv[sl]
        sv[sl] = jnp.where(s > _SCORE_THRESH, s, -1.0)
        av[sl] = (x2v[sl] - x1v[sl]) * (y2v[sl] - y1v[sl])

    def round_body(r, carry):
        bx1, by1, bx2, by2, ba = carry

        acc0 = (jnp.full((_L,), neg, jnp.float32),
                jnp.full((_L,), 2 ** 30, jnp.int32))

        iz = r * 0
        accs = (acc0, acc0, acc0, acc0)
        for i in range(0, per, 4 * _L):
            out = []
            for k in range(4):
                accv, acci = accs[k]
                sl = pl.ds(iz + (i + k * _L), _L)
                xx1 = x1v[sl]
                yy1 = y1v[sl]
                xx2 = x2v[sl]
                yy2 = y2v[sl]
                ar = av[sl]
                s = sv[sl]
                w = jnp.maximum(
                    jnp.minimum(bx2, xx2) - jnp.maximum(bx1, xx1), 0.0)
                h = jnp.maximum(
                    jnp.minimum(by2, yy2) - jnp.maximum(by1, yy1), 0.0)
                inter = w * h
                iou = inter / (ba + ar - inter + 1e-9)
                s = jnp.where(iou > _NMS_THRESH, -1.0, s)
                sv[sl] = s
                idx = i + (k * _L + lanei)
                upd = (s > accv) | ((s == accv) & (idx < acci))
                accv = jnp.where(upd, s, accv)
                acci = jnp.where(upd, idx, acci)
                out.append((accv, acci))
            accs = tuple(out)

        (accv, acci) = accs[0]
        for k in range(1, 4):
            v2, i2 = accs[k]
            better = (v2 > accv) | ((v2 == accv) & (i2 < acci))
            accv = jnp.where(better, v2, accv)
            acci = jnp.where(better, i2, acci)
        mvec = _bcast_max(accv, lanei)
        limin = jnp.where(accv >= mvec, acci, jnp.int32(2 ** 30))
        for k in (8, 4, 2, 1):
            limin = jnp.minimum(limin, _permute(limin, lanei ^ k))
        cx1 = plsc.load_gather(x1v, [limin])
        cy1 = plsc.load_gather(y1v, [limin])
        cx2 = plsc.load_gather(x2v, [limin])
        cy2 = plsc.load_gather(y2v, [limin])
        car = (cx2 - cx1) * (cy2 - cy1)
        pack = jnp.where(lanei == 0, cx1,
               jnp.where(lanei == 1, cy1,
               jnp.where(lanei == 2, cx2,
               jnp.where(lanei == 3, cy2,
               jnp.where(lanei == 4, mvec,
               jnp.where(lanei == 5, car, 0.0))))))
        stage[...] = pack
        off = (r & 1) * (_NS * _L)
        pltpu.sync_copy(stage, sh_pack.at[pl.ds(off + tid * _L, _L)])
        plsc.subcore_barrier()
        pltpu.sync_copy(sh_pack.at[pl.ds(off, _NS * _L)], loc_pack)

        rows = [loc_pack[pl.ds(t * _L, _L)] for t in range(_NS)]
        pairs = [(_permute(p, zi + 4), p) for p in rows]
        while len(pairs) > 1:
            nxt = []
            for a in range(0, len(pairs), 2):
                (vl, pl_), (vr, pr) = pairs[a], pairs[a + 1]
                upd = vr > vl
                nxt.append((jnp.where(upd, vr, vl),
                            jnp.where(upd, pr, pl_)))
            pairs = nxt
        gv, g = pairs[0]

        nbx1 = _permute(g, zi)
        nby1 = _permute(g, zi + 1)
        nbx2 = _permute(g, zi + 2)
        nby2 = _permute(g, zi + 3)
        nba = (nbx2 - nbx1) * (nby2 - nby1)

        rowv = jnp.where((lanei <= 4) & (gv > 0.0), g, 0.0)
        out_stage[pl.ds(r * _L, _L)] = rowv

        return nbx1, nby1, nbx2, nby2, nba

    z = jnp.zeros((_L,), jnp.float32)
    lax.fori_loop(0, _MAX_DET, round_body, (z, z, z, z, z))

    @pl.when(jnp.logical_and(cid == 0, tid == 0))
    def _():
        pltpu.sync_copy(out_stage, outh)


def kernel(boxes, scores):
    n = boxes.shape[0]
    npad = ((n + _NS * _L - 1) // (_NS * _L)) * (_NS * _L)
    pad = npad - n
    b = jnp.pad(boxes, ((0, pad), (0, 0)))
    s = jnp.pad(scores, (0, pad))
    per = npad // _NS
    mesh = plsc.VectorSubcoreMesh(core_axis_name="c", subcore_axis_name="s", num_cores=1)
    f = functools.partial(
        pl.kernel,
        out_type=jax.ShapeDtypeStruct((_MAX_DET * _L,), jnp.float32),
        mesh=mesh,
        compiler_params=pltpu.CompilerParams(needs_layout_passes=False),
        scratch_types=[
            pltpu.VMEM((per + 32,), jnp.float32),
            pltpu.VMEM((per + 32,), jnp.float32),
            pltpu.VMEM((per + 32,), jnp.float32),
            pltpu.VMEM((per + 32,), jnp.float32),
            pltpu.VMEM((per + 32,), jnp.float32),
            pltpu.VMEM((per + 32,), jnp.float32),
            pltpu.VMEM((_L,), jnp.float32),
            pltpu.VMEM((_NS * _L,), jnp.float32),
            pltpu.VMEM((_MAX_DET * _L,), jnp.float32),
            pltpu.VMEM_SHARED((2 * _NS * _L,), jnp.float32),
            pltpu.SemaphoreType.DMA,
        ],
    )(_nms_sc_body)
    out = f(b[:, 0], b[:, 1], b[:, 2], b[:, 3], s)
    return out.reshape(_MAX_DET, _L)[:, :5]
